# baseline (speedup 1.0000x reference)
import functools

import jax
import jax.numpy as jnp
from jax import lax
from jax.experimental import pallas as pl
from jax.experimental.pallas import tpu as pltpu
from jax.experimental.pallas import tpu_sc as plsc


def _make_sc_gather(V, D, Bk, n_workers, chunk):
    b_per_w = Bk // n_workers
    n_chunks = b_per_w // chunk
    mesh = plsc.VectorSubcoreMesh(core_axis_name="c", subcore_axis_name="s")

    @functools.partial(
        pl.kernel,
        mesh=mesh,
        out_type=jax.ShapeDtypeStruct((Bk, D), jnp.float32),
        scratch_types=[
            pltpu.VMEM((b_per_w,), jnp.int32),
            pltpu.VMEM((chunk, D), jnp.float32),
            pltpu.VMEM((chunk, D), jnp.float32),
            pltpu.SemaphoreType.DMA,
            pltpu.SemaphoreType.DMA,
        ],
    )
    def gather_kernel(tab_hbm, idx_hbm, out_hbm, idx_v, buf0, buf1, sem0, sem1):
        n_cores = 2
        wid = lax.axis_index("s") * n_cores + lax.axis_index("c")
        base = wid * b_per_w
        pltpu.sync_copy(idx_hbm.at[pl.ds(base, b_per_w)], idx_v)

        bufs = (buf0, buf1)
        sems = (sem0, sem1)
        gathers = []
        for i in range(n_chunks):
            g = pltpu.async_copy(
                tab_hbm.at[idx_v.at[pl.ds(i * chunk, chunk)]],
                bufs[i % 2],
                sems[i % 2],
            )
            gathers.append(g)
            if i >= 1:
                gathers[i - 1].wait()
                pltpu.sync_copy(
                    bufs[(i - 1) % 2],
                    out_hbm.at[pl.ds(base + (i - 1) * chunk, chunk)],
                )
        gathers[n_chunks - 1].wait()
        pltpu.sync_copy(
            bufs[(n_chunks - 1) % 2],
            out_hbm.at[pl.ds(base + (n_chunks - 1) * chunk, chunk)],
        )

    return gather_kernel


def _tcproj_body(ns_b, Bt, x_ref, w_ref, b_ref, o_ref):
    y = (
        jnp.dot(x_ref[...], w_ref[...], preferred_element_type=jnp.float32)
        + b_ref[...]
    )
    o_ref[...] = y.reshape(ns_b, Bt, y.shape[-1])


def _project_chunk(flat_k, W, b2d, acc, S, Bt, s0, ns, ns_b):
    H = flat_k.shape[1]
    D = W.shape[1]
    grid = (ns // ns_b,)
    s_blk0 = s0 // ns_b
    in_specs = [
        pl.BlockSpec((ns_b * Bt, H), lambda i: (i, 0)),
        pl.BlockSpec((H, D), lambda i: (0, 0)),
        pl.BlockSpec((1, D), lambda i: (0, 0)),
    ]
    operands = [flat_k, W, b2d]
    kwargs = {}
    if acc is not None:
        in_specs.append(pl.BlockSpec(memory_space=pl.ANY))
        operands.append(acc)
        kwargs["input_output_aliases"] = {3: 0}

    def body(x_ref, w_ref, b_ref, *rest):
        o_ref = rest[-1]
        _tcproj_body(ns_b, Bt, x_ref, w_ref, b_ref, o_ref)

    return pl.pallas_call(
        body,
        grid=grid,
        in_specs=in_specs,
        out_specs=pl.BlockSpec((ns_b, Bt, D), lambda i: (s_blk0 + i, 0, 0)),
        out_shape=jax.ShapeDtypeStruct((S, Bt, D), jnp.float32),
        **kwargs,
    )(*operands)


def kernel(input_ids, table, W, b):
    Bt, S = input_ids.shape
    V, H = table.shape
    D = W.shape[1]

    K = 5
    ns = S // K
    ns_b = 2
    b2d = b.reshape(1, D)

    idxT = input_ids.T.astype(jnp.int32)
    sc_gather = _make_sc_gather(V, H, ns * Bt, n_workers=32, chunk=256)

    flats = [
        sc_gather(
            table,
            lax.slice_in_dim(idxT, k * ns, (k + 1) * ns).reshape(ns * Bt),
        )
        for k in range(K)
    ]

    out = None
    for k in range(K):
        out = _project_chunk(flats[k], W, b2d, out, S, Bt, k * ns, ns, ns_b)
    return out.transpose(1, 0, 2)


# speedup vs baseline: 1.3449x; 1.3449x over previous
import functools

import jax
import jax.numpy as jnp
from jax import lax
from jax.experimental import pallas as pl
from jax.experimental.pallas import tpu as pltpu
from jax.experimental.pallas import tpu_sc as plsc


def _proj_body(t_ref, w_ref, b_ref, o_ref):
    o_ref[...] = (
        jnp.dot(t_ref[...], w_ref[...], preferred_element_type=jnp.float32)
        + b_ref[...]
    )


def _project_table(table, W, b2d, block_rows):
    V, H = table.shape
    D = W.shape[1]
    grid = (V // block_rows,)
    return pl.pallas_call(
        _proj_body,
        grid=grid,
        in_specs=[
            pl.BlockSpec((block_rows, H), lambda i: (i, 0)),
            pl.BlockSpec((H, D), lambda i: (0, 0)),
            pl.BlockSpec((1, D), lambda i: (0, 0)),
        ],
        out_specs=pl.BlockSpec((block_rows, D), lambda i: (i, 0)),
        out_shape=jax.ShapeDtypeStruct((V, D), jnp.float32),
    )(table, W, b2d)


def _make_sc_gather(V, D, B, n_workers, chunk):
    b_per_w = B // n_workers
    n_chunks = b_per_w // chunk
    mesh = plsc.VectorSubcoreMesh(core_axis_name="c", subcore_axis_name="s")

    @functools.partial(
        pl.kernel,
        mesh=mesh,
        out_type=jax.ShapeDtypeStruct((B, D), jnp.float32),
        scratch_types=[
            pltpu.VMEM((b_per_w,), jnp.int32),
            pltpu.VMEM((chunk, D), jnp.float32),
            pltpu.VMEM((chunk, D), jnp.float32),
            pltpu.VMEM((chunk, D), jnp.float32),
            pltpu.SemaphoreType.DMA,
            pltpu.SemaphoreType.DMA,
            pltpu.SemaphoreType.DMA,
            pltpu.SemaphoreType.DMA,
            pltpu.SemaphoreType.DMA,
            pltpu.SemaphoreType.DMA,
        ],
    )
    def gather_kernel(tab_hbm, idx_hbm, out_hbm, idx_v, buf0, buf1, buf2,
                      g0, g1, g2, w0, w1, w2):
        n_cores = 2
        wid = lax.axis_index("s") * n_cores + lax.axis_index("c")
        base = wid * b_per_w
        pltpu.sync_copy(idx_hbm.at[pl.ds(base, b_per_w)], idx_v)

        bufs = (buf0, buf1, buf2)
        gsems = (g0, g1, g2)
        wsems = (w0, w1, w2)
        gathers = []
        wbs = []
        for i in range(n_chunks):
            if i >= 3:
                wbs[i - 3].wait()
            gathers.append(pltpu.async_copy(
                tab_hbm.at[idx_v.at[pl.ds(i * chunk, chunk)]],
                bufs[i % 3],
                gsems[i % 3],
            ))
            if i >= 1:
                gathers[i - 1].wait()
                wbs.append(pltpu.async_copy(
                    bufs[(i - 1) % 3],
                    out_hbm.at[pl.ds(base + (i - 1) * chunk, chunk)],
                    wsems[(i - 1) % 3],
                ))
        gathers[n_chunks - 1].wait()
        wbs.append(pltpu.async_copy(
            bufs[(n_chunks - 1) % 3],
            out_hbm.at[pl.ds(base + (n_chunks - 1) * chunk, chunk)],
            wsems[(n_chunks - 1) % 3],
        ))
        for j in range(max(0, n_chunks - 3), n_chunks):
            wbs[j].wait()

    return gather_kernel


def kernel(input_ids, table, W, b):
    Bt, S = input_ids.shape
    V, H = table.shape
    D = W.shape[1]
    B = Bt * S

    proj = _project_table(table, W, b.reshape(1, D), block_rows=20000)

    idx = input_ids.T.reshape(B).astype(jnp.int32)
    flat = _make_sc_gather(V, D, B, n_workers=32, chunk=256)(proj, idx)
    return flat.reshape(S, Bt, D).transpose(1, 0, 2)
